# baseline (speedup 1.0000x reference)
import functools

import jax
import jax.numpy as jnp
from jax.experimental import pallas as pl
from jax.experimental.pallas import tpu as pltpu

N = 2708
NP = 2816
F = 1433
H = 128
NL = 5
R = 256
NB = NP // R
CAT = NL * H
INV = 1.0 / (1.0 + 1e-5) ** 0.5


def _body(adj_ref, x_ref, w1_ref, w2_ref, w3_ref, w4_ref, w5_ref,
          bgc_ref, wa_ref, ba_ref,
          wh1_ref, hv_ref, wh2_ref, wh3_ref, cat_ref, out_ref,
          p_ref, s_ref, xb_ref, adj_s_ref):
    p = pl.program_id(0)
    r = pl.program_id(1)
    row = r * R

    @pl.when(p == 0)
    def _load():
        @pl.when(r == 0)
        def _zero_pad_cols():
            adj_s_ref[:, N:] = jnp.zeros((NP, NP - N), jnp.bfloat16)

        rmask = jax.lax.broadcasted_iota(jnp.int32, (R, N), 0) < (N - row)
        adj_s_ref[pl.ds(row, R), :N] = jnp.where(
            rmask, adj_ref[...], 0.0).astype(jnp.bfloat16)

        xmask = jax.lax.broadcasted_iota(jnp.int32, (R, F), 0) < (N - row)
        xv = jnp.where(xmask, x_ref[...], 0.0).astype(jnp.bfloat16)
        ws = [w1_ref, w2_ref, w3_ref, w4_ref, w5_ref]
        s_ref[0, pl.ds(row, R), :] = jnp.dot(
            xv, ws[0][...], preferred_element_type=jnp.float32).astype(
                jnp.bfloat16)
        for l in range(1, NL):
            p_ref[l - 1, pl.ds(row, R), :] = jnp.dot(
                xv, ws[l][:F, :], preferred_element_type=jnp.float32)

    @pl.when(p > 0)
    def _layer():
        l = p - 1
        cur = jax.lax.rem(l, 2)
        a_blk = adj_s_ref[pl.ds(row, R), :]
        acc = jnp.dot(a_blk, s_ref[cur], preferred_element_type=jnp.float32)
        xi = acc + bgc_ref[l, 0:1, :]
        xi = jnp.where(l == 0, jnp.maximum(xi, 0.0), xi)
        cat_ref[...] = xi

        wrs = [w2_ref, w3_ref, w4_ref, w5_ref]
        for ll in range(NL - 1):
            @pl.when(l == ll)
            def _advance(ll=ll):
                xb_ref[ll, pl.ds(row, R), :] = xi
                xcat = (jnp.concatenate(
                    [xb_ref[j, pl.ds(row, R), :] for j in range(ll + 1)],
                    axis=1) if ll > 0 else xi).astype(jnp.bfloat16)
                s_ref[1 - cur, pl.ds(row, R), :] = (
                    p_ref[ll, pl.ds(row, R), :]
                    + jnp.dot(xcat, wrs[ll][F:, :],
                              preferred_element_type=jnp.float32)
                ).astype(jnp.bfloat16)

        @pl.when(l == NL - 1)
        def _head():
            catr = jnp.concatenate(
                [xb_ref[j, pl.ds(row, R), :] for j in range(NL - 1)] + [xi],
                axis=1)
            logits = jnp.dot(catr.astype(jnp.bfloat16), wa_ref[...],
                             preferred_element_type=jnp.float32) + ba_ref[0:1, :]
            m = jnp.max(logits, axis=1, keepdims=True)
            e = jnp.exp(logits - m)
            aw = e / jnp.sum(e, axis=1, keepdims=True)
            att = (catr * aw).astype(jnp.bfloat16)
            h = jnp.dot(att, wh1_ref[...],
                        preferred_element_type=jnp.float32) + hv_ref[0:1, :]
            h = jnp.maximum(hv_ref[1:2, :] * (h * INV) + hv_ref[2:3, :], 0.0)
            h2 = jnp.dot(h.astype(jnp.bfloat16), wh2_ref[...],
                         preferred_element_type=jnp.float32) + hv_ref[3:4, :]
            h2 = jnp.maximum(hv_ref[4:5, :] * (h2 * INV) + hv_ref[5:6, :], 0.0)
            lg = jnp.dot(h2.astype(jnp.bfloat16), wh3_ref[...],
                         preferred_element_type=jnp.float32) + hv_ref[6:7, :]
            col = jax.lax.broadcasted_iota(jnp.int32, (R, H), 1)
            mask = col < 7
            lgm = jnp.where(mask, lg, -1e30)
            mm = jnp.max(lgm, axis=1, keepdims=True)
            ee = jnp.where(mask, jnp.exp(lg - mm), 0.0)
            out_ref[...] = lg - mm - jnp.log(jnp.sum(ee, axis=1, keepdims=True))


def _pad_vec(v, n):
    return jnp.pad(v, (0, n - v.shape[0]))


@functools.partial(jax.jit)
def kernel(x, adj, W1, b1, W2, b2, W3, b3, W4, b4, W5, b5, Wa, ba,
           Wh1, bh1, g1, be1, Wh2, bh2, g2, be2, Wh3, bh3):
    bgc = jnp.pad(jnp.stack([b1, b2, b3, b4, b5]).reshape(NL, 1, H),
                  ((0, 0), (0, 7), (0, 0)))
    ba_p = jnp.pad(ba.reshape(1, CAT), ((0, 7), (0, 0)))
    hv = jnp.pad(jnp.stack([bh1, g1, be1, _pad_vec(bh2, H), _pad_vec(g2, H),
                            _pad_vec(be2, H), _pad_vec(bh3, H)]),
                 ((0, 1), (0, 0)))
    wh2p = jnp.pad(Wh2, ((0, 0), (0, H - Wh2.shape[1])))
    wh3p = jnp.pad(Wh3, ((0, H - Wh3.shape[0]), (0, H - Wh3.shape[1])))
    bf = jnp.bfloat16
    W1, W2, W3, W4, W5 = (w.astype(bf) for w in (W1, W2, W3, W4, W5))
    Wa, Wh1, wh2p, wh3p = (w.astype(bf) for w in (Wa, Wh1, wh2p, wh3p))

    full = lambda shape: pl.BlockSpec(shape, lambda p, r: (0,) * len(shape))
    stream = pl.BlockSpec((R, N), lambda p, r: (jnp.where(p == 0, r, 0), 0))
    streamx = pl.BlockSpec((R, F), lambda p, r: (jnp.where(p == 0, r, 0), 0))

    cat, outp = pl.pallas_call(
        _body,
        grid=(NL + 1, NB),
        in_specs=[
            stream,
            streamx,
            full((F, H)), full((F + H, H)), full((F + 2 * H, H)),
            full((F + 3 * H, H)), full((F + 4 * H, H)),
            full((NL, 8, H)),
            full((CAT, CAT)),
            full((8, CAT)),
            full((CAT, H)),
            full((8, H)),
            full((H, H)),
            full((H, H)),
        ],
        out_specs=[
            pl.BlockSpec((R, H), lambda p, r: (jnp.where(p == 0, 0, r),
                                               jnp.maximum(p - 1, 0))),
            pl.BlockSpec((R, H), lambda p, r: (jnp.where(p == NL, r, 0), 0)),
        ],
        out_shape=[
            jax.ShapeDtypeStruct((N, CAT), jnp.float32),
            jax.ShapeDtypeStruct((N, H), jnp.float32),
        ],
        scratch_shapes=[
            pltpu.VMEM((NL - 1, NP, H), jnp.float32),
            pltpu.VMEM((2, NP, H), jnp.bfloat16),
            pltpu.VMEM((NL - 1, NP, H), jnp.float32),
            pltpu.VMEM((NP, NP), jnp.bfloat16),
        ],
        compiler_params=pltpu.CompilerParams(
            dimension_semantics=("arbitrary", "arbitrary"),
            vmem_limit_bytes=128 * 1024 * 1024),
    )(adj, x, W1, W2, W3, W4, W5, bgc, Wa, ba_p, Wh1, hv, wh2p, wh3p)

    features = cat.reshape(N, NL, H)
    return outp[:, :7], features


# speedup vs baseline: 1.0239x; 1.0239x over previous
import functools

import jax
import jax.numpy as jnp
from jax.experimental import pallas as pl
from jax.experimental.pallas import tpu as pltpu

N = 2708
NP = 2816
F = 1433
H = 128
NL = 5
R = 256
NB = NP // R
CAT = NL * H
INV = 1.0 / (1.0 + 1e-5) ** 0.5


def _body(adj_ref, x_ref, wcat_ref, wr1_ref, wr2_ref, wr3_ref, wr4_ref,
          bgc_ref, wa_ref, ba_ref,
          wh1_ref, hv_ref, wh2_ref, wh3_ref, cat_ref, out_ref,
          p_ref, cs_ref, adj_s_ref):
    p = pl.program_id(0)
    r = pl.program_id(1)
    row = r * R

    @pl.when(p == 0)
    def _load():
        @pl.when(r == 0)
        def _zero_pad_cols():
            adj_s_ref[:, N:] = jnp.zeros((NP, NP - N), jnp.float32)

        rmask = jax.lax.broadcasted_iota(jnp.int32, (R, N), 0) < (N - row)
        adj_s_ref[pl.ds(row, R), :N] = jnp.where(rmask, adj_ref[...], 0.0)

        xmask = jax.lax.broadcasted_iota(jnp.int32, (R, F), 0) < (N - row)
        xv = jnp.where(xmask, x_ref[...], 0.0)
        proj = jnp.dot(xv, wcat_ref[...], preferred_element_type=jnp.float32)
        for l in range(NL):
            p_ref[l, pl.ds(row, R), :] = proj[:, l * H:(l + 1) * H]

    @pl.when(p > 0)
    def _layer():
        l = p - 1
        a_blk = adj_s_ref[pl.ds(row, R), :]
        acc = jnp.dot(a_blk, p_ref[l], preferred_element_type=jnp.float32)
        xi = acc + bgc_ref[l, 0:1, :]
        xi = jnp.where(l == 0, jnp.maximum(xi, 0.0), xi)
        cat_ref[...] = xi

        wrs = [wr1_ref, wr2_ref, wr3_ref, wr4_ref]
        for ll in range(NL - 1):
            @pl.when(l == ll)
            def _scatter(ll=ll):
                cs_ref[pl.ds(row, R), ll * H:(ll + 1) * H] = xi
                upd = jnp.dot(xi, wrs[ll][...],
                              preferred_element_type=jnp.float32)
                for k, m in enumerate(range(ll + 1, NL)):
                    p_ref[m, pl.ds(row, R), :] = (
                        p_ref[m, pl.ds(row, R), :]
                        + upd[:, k * H:(k + 1) * H])

        @pl.when(l == NL - 1)
        def _head():
            catr = jnp.concatenate(
                [cs_ref[pl.ds(row, R), :], xi], axis=1)
            logits = jnp.dot(catr, wa_ref[...],
                             preferred_element_type=jnp.float32) + ba_ref[0:1, :]
            m = jnp.max(logits, axis=1, keepdims=True)
            e = jnp.exp(logits - m)
            aw = e / jnp.sum(e, axis=1, keepdims=True)
            att = catr * aw
            h = jnp.dot(att, wh1_ref[...],
                        preferred_element_type=jnp.float32) + hv_ref[0:1, :]
            h = jnp.maximum(hv_ref[1:2, :] * (h * INV) + hv_ref[2:3, :], 0.0)
            h2 = jnp.dot(h, wh2_ref[...],
                         preferred_element_type=jnp.float32) + hv_ref[3:4, :]
            h2 = jnp.maximum(hv_ref[4:5, :] * (h2 * INV) + hv_ref[5:6, :], 0.0)
            lg = jnp.dot(h2, wh3_ref[...],
                         preferred_element_type=jnp.float32) + hv_ref[6:7, :]
            col = jax.lax.broadcasted_iota(jnp.int32, (R, H), 1)
            mask = col < 7
            lgm = jnp.where(mask, lg, -1e30)
            mm = jnp.max(lgm, axis=1, keepdims=True)
            ee = jnp.where(mask, jnp.exp(lg - mm), 0.0)
            out_ref[...] = lg - mm - jnp.log(jnp.sum(ee, axis=1, keepdims=True))


def _pad_vec(v, n):
    return jnp.pad(v, (0, n - v.shape[0]))


@functools.partial(jax.jit)
def kernel(x, adj, W1, b1, W2, b2, W3, b3, W4, b4, W5, b5, Wa, ba,
           Wh1, bh1, g1, be1, Wh2, bh2, g2, be2, Wh3, bh3):
    ws = [W1, W2, W3, W4, W5]
    wcat = jnp.concatenate([w[:F, :] for w in ws], axis=1)
    wrows = [jnp.concatenate([ws[m][F + ll * H:F + (ll + 1) * H, :]
                              for m in range(ll + 1, NL)], axis=1)
             for ll in range(NL - 1)]
    bgc = jnp.pad(jnp.stack([b1, b2, b3, b4, b5]).reshape(NL, 1, H),
                  ((0, 0), (0, 7), (0, 0)))
    ba_p = jnp.pad(ba.reshape(1, CAT), ((0, 7), (0, 0)))
    hv = jnp.pad(jnp.stack([bh1, g1, be1, _pad_vec(bh2, H), _pad_vec(g2, H),
                            _pad_vec(be2, H), _pad_vec(bh3, H)]),
                 ((0, 1), (0, 0)))
    wh2p = jnp.pad(Wh2, ((0, 0), (0, H - Wh2.shape[1])))
    wh3p = jnp.pad(Wh3, ((0, H - Wh3.shape[0]), (0, H - Wh3.shape[1])))

    full = lambda shape: pl.BlockSpec(shape, lambda p, r: (0,) * len(shape))
    stream = pl.BlockSpec((R, N), lambda p, r: (jnp.where(p == 0, r, 0), 0))
    streamx = pl.BlockSpec((R, F), lambda p, r: (jnp.where(p == 0, r, 0), 0))

    cat, outp = pl.pallas_call(
        _body,
        grid=(NL + 1, NB),
        in_specs=[
            stream,
            streamx,
            full((F, CAT)),
            full((H, 4 * H)), full((H, 3 * H)), full((H, 2 * H)),
            full((H, H)),
            full((NL, 8, H)),
            full((CAT, CAT)),
            full((8, CAT)),
            full((CAT, H)),
            full((8, H)),
            full((H, H)),
            full((H, H)),
        ],
        out_specs=[
            pl.BlockSpec((R, H), lambda p, r: (jnp.where(p == 0, 0, r),
                                               jnp.maximum(p - 1, 0))),
            pl.BlockSpec((R, H), lambda p, r: (jnp.where(p == NL, r, 0), 0)),
        ],
        out_shape=[
            jax.ShapeDtypeStruct((N, CAT), jnp.float32),
            jax.ShapeDtypeStruct((N, H), jnp.float32),
        ],
        scratch_shapes=[
            pltpu.VMEM((NL, NP, H), jnp.float32),
            pltpu.VMEM((NP, (NL - 1) * H), jnp.float32),
            pltpu.VMEM((NP, NP), jnp.float32),
        ],
        compiler_params=pltpu.CompilerParams(
            dimension_semantics=("arbitrary", "arbitrary"),
            vmem_limit_bytes=128 * 1024 * 1024),
    )(adj, x, wcat, wrows[0], wrows[1], wrows[2], wrows[3],
      bgc, Wa, ba_p, Wh1, hv, wh2p, wh3p)

    features = cat.reshape(N, NL, H)
    return outp[:, :7], features


# speedup vs baseline: 1.0923x; 1.0668x over previous
import functools

import jax
import jax.numpy as jnp
from jax.experimental import pallas as pl
from jax.experimental.pallas import tpu as pltpu

N = 2708
NP = 2816
F = 1433
H = 128
NL = 5
R = 256
NB = NP // R
CAT = NL * H
INV = 1.0 / (1.0 + 1e-5) ** 0.5


def _body(adj_ref, x_ref, w1_ref, w2_ref, w3_ref, w4_ref, w5_ref,
          bgc_ref, wa_ref, ba_ref,
          wh1_ref, hv_ref, wh2_ref, wh3_ref, cat_ref, out_ref,
          p_ref, s_ref, xb_ref, adj_s_ref):
    p = pl.program_id(0)
    r = pl.program_id(1)
    row = r * R

    @pl.when(p == 0)
    def _load():
        @pl.when(r == 0)
        def _zero_pad_cols():
            adj_s_ref[:, N:] = jnp.zeros((NP, NP - N), jnp.float32)

        rmask = jax.lax.broadcasted_iota(jnp.int32, (R, N), 0) < (N - row)
        adj_s_ref[pl.ds(row, R), :N] = jnp.where(rmask, adj_ref[...], 0.0)

        xmask = jax.lax.broadcasted_iota(jnp.int32, (R, F), 0) < (N - row)
        xv = jnp.where(xmask, x_ref[...], 0.0)
        ws = [w1_ref, w2_ref, w3_ref, w4_ref, w5_ref]
        s_ref[0, pl.ds(row, R), :] = jnp.dot(
            xv, ws[0][...], preferred_element_type=jnp.float32)
        for l in range(1, NL):
            p_ref[l - 1, pl.ds(row, R), :] = jnp.dot(
                xv, ws[l][:F, :], preferred_element_type=jnp.float32)

    @pl.when(p > 0)
    def _layer():
        l = p - 1
        cur = jax.lax.rem(l, 2)
        a_blk = adj_s_ref[pl.ds(row, R), :]
        acc = jnp.dot(a_blk, s_ref[cur], preferred_element_type=jnp.float32)
        xi = acc + bgc_ref[l, 0:1, :]
        xi = jnp.where(l == 0, jnp.maximum(xi, 0.0), xi)
        cat_ref[...] = xi

        wrs = [w2_ref, w3_ref, w4_ref, w5_ref]
        for ll in range(NL - 1):
            @pl.when(l == ll)
            def _advance(ll=ll):
                xb_ref[ll, pl.ds(row, R), :] = xi
                xcat = jnp.concatenate(
                    [xb_ref[j, pl.ds(row, R), :] for j in range(ll + 1)],
                    axis=1) if ll > 0 else xi
                s_ref[1 - cur, pl.ds(row, R), :] = (
                    p_ref[ll, pl.ds(row, R), :]
                    + jnp.dot(xcat, wrs[ll][F:, :],
                              preferred_element_type=jnp.float32))

        @pl.when(l == NL - 1)
        def _head():
            catr = jnp.concatenate(
                [xb_ref[j, pl.ds(row, R), :] for j in range(NL - 1)] + [xi],
                axis=1)
            logits = jnp.dot(catr, wa_ref[...],
                             preferred_element_type=jnp.float32) + ba_ref[0:1, :]
            m = jnp.max(logits, axis=1, keepdims=True)
            e = jnp.exp(logits - m)
            aw = e / jnp.sum(e, axis=1, keepdims=True)
            att = catr * aw
            h = jnp.dot(att, wh1_ref[...],
                        preferred_element_type=jnp.float32) + hv_ref[0:1, :]
            h = jnp.maximum(hv_ref[1:2, :] * (h * INV) + hv_ref[2:3, :], 0.0)
            h2 = jnp.dot(h, wh2_ref[...],
                         preferred_element_type=jnp.float32) + hv_ref[3:4, :]
            h2 = jnp.maximum(hv_ref[4:5, :] * (h2 * INV) + hv_ref[5:6, :], 0.0)
            lg = jnp.dot(h2, wh3_ref[...],
                         preferred_element_type=jnp.float32) + hv_ref[6:7, :]
            col = jax.lax.broadcasted_iota(jnp.int32, (R, H), 1)
            mask = col < 7
            lgm = jnp.where(mask, lg, -1e30)
            mm = jnp.max(lgm, axis=1, keepdims=True)
            ee = jnp.where(mask, jnp.exp(lg - mm), 0.0)
            out_ref[...] = lg - mm - jnp.log(jnp.sum(ee, axis=1, keepdims=True))


def _pad_vec(v, n):
    return jnp.pad(v, (0, n - v.shape[0]))


@functools.partial(jax.jit)
def kernel(x, adj, W1, b1, W2, b2, W3, b3, W4, b4, W5, b5, Wa, ba,
           Wh1, bh1, g1, be1, Wh2, bh2, g2, be2, Wh3, bh3):
    bgc = jnp.pad(jnp.stack([b1, b2, b3, b4, b5]).reshape(NL, 1, H),
                  ((0, 0), (0, 7), (0, 0)))
    ba_p = jnp.pad(ba.reshape(1, CAT), ((0, 7), (0, 0)))
    hv = jnp.pad(jnp.stack([bh1, g1, be1, _pad_vec(bh2, H), _pad_vec(g2, H),
                            _pad_vec(be2, H), _pad_vec(bh3, H)]),
                 ((0, 1), (0, 0)))
    wh2p = jnp.pad(Wh2, ((0, 0), (0, H - Wh2.shape[1])))
    wh3p = jnp.pad(Wh3, ((0, H - Wh3.shape[0]), (0, H - Wh3.shape[1])))

    full = lambda shape: pl.BlockSpec(shape, lambda p, r: (0,) * len(shape))
    stream = pl.BlockSpec((R, N), lambda p, r: (jnp.where(p == 0, r, 0), 0))
    streamx = pl.BlockSpec((R, F), lambda p, r: (jnp.where(p == 0, r, 0), 0))

    cat, outp = pl.pallas_call(
        _body,
        grid=(NL + 1, NB),
        in_specs=[
            stream,
            streamx,
            full((F, H)), full((F + H, H)), full((F + 2 * H, H)),
            full((F + 3 * H, H)), full((F + 4 * H, H)),
            full((NL, 8, H)),
            full((CAT, CAT)),
            full((8, CAT)),
            full((CAT, H)),
            full((8, H)),
            full((H, H)),
            full((H, H)),
        ],
        out_specs=[
            pl.BlockSpec((R, H), lambda p, r: (jnp.where(p == 0, 0, r),
                                               jnp.maximum(p - 1, 0))),
            pl.BlockSpec((R, H), lambda p, r: (jnp.where(p == NL, r, 0), 0)),
        ],
        out_shape=[
            jax.ShapeDtypeStruct((N, CAT), jnp.float32),
            jax.ShapeDtypeStruct((N, H), jnp.float32),
        ],
        scratch_shapes=[
            pltpu.VMEM((NL - 1, NP, H), jnp.float32),
            pltpu.VMEM((2, NP, H), jnp.float32),
            pltpu.VMEM((NL - 1, NP, H), jnp.float32),
            pltpu.VMEM((NP, NP), jnp.float32),
        ],
        compiler_params=pltpu.CompilerParams(
            dimension_semantics=("arbitrary", "arbitrary"),
            vmem_limit_bytes=128 * 1024 * 1024),
    )(adj, x, W1, W2, W3, W4, W5, bgc, Wa, ba_p, Wh1, hv, wh2p, wh3p)

    features = cat.reshape(N, NL, H)
    return outp[:, :7], features
